# baseline (speedup 1.0000x reference)
import functools

import jax
import jax.numpy as jnp
from jax import lax
from jax.experimental import pallas as pl
from jax.experimental.pallas import tpu as pltpu
from jax.experimental.pallas import tpu_sc as plsc

_B = 16384
_D = 64
_V = 100000
_EPS = 1e-5
_H = _B // 2
_NT = 16
_FPT = _D // _NT
_NV = _H // 16
_SLICE = _H // _NT
_PASS = ((0, 33336), (33336, 33336), (66672, _V - 66672))
_NP = len(_PASS)
_BUFW = 33336


def _rsqrt(v):
    i = lax.bitcast_convert_type(v, jnp.int32)
    i = jnp.int32(0x5F3759DF) - lax.shift_right_arithmetic(i, 1)
    y = lax.bitcast_convert_type(i, jnp.float32)
    hv = jnp.float32(0.5) * v
    for _ in range(2):
        y = y * (jnp.float32(1.5) - hv * y * y)
    return y


def _bcast_lane(vec, lane):
    idx = jnp.full((16,), lane, dtype=jnp.int32)
    return vec.at[idx].get(mode="promise_in_bounds")


def _body(bbase, x_hbm, tab_hbm, gamma_hbm, beta_hbm, out_hbm,
          idx_v, buf0, buf1, g_v, sum_v, sq_v, part_v,
          gam_v, bet_v, sh_part, sh_stats, sem0, sem1, wsem, psem):
    sid = lax.axis_index("s")

    pltpu.sync_copy(x_hbm.at[pl.ds(bbase, _H)], idx_v)
    pltpu.sync_copy(gamma_hbm, gam_v)
    pltpu.sync_copy(beta_hbm, bet_v)

    bufs = (buf0, buf1)
    sems = (sem0, sem1)
    lanes = lax.iota(jnp.int32, 16)
    inv_d = jnp.float32(1.0 / _D)

    def start(i):
        k, p = divmod(i, _NP)
        off, sz = _PASS[p]
        f = sid * _FPT + k
        return pltpu.async_copy(tab_hbm.at[f, pl.ds(off, sz)],
                                bufs[i % 2].at[pl.ds(0, sz)], sems[i % 2])

    def run_pass(buf, p):
        off, sz = _PASS[p]
        lo = jnp.int32(off)

        def vbody(v):
            base = v * 16
            iv = idx_v[pl.ds(base, 16)]
            loc = iv - lo
            ul = lax.bitcast_convert_type(loc, jnp.uint32)
            m = ul < jnp.uint32(sz)
            locc = lax.bitcast_convert_type(
                jnp.minimum(ul, jnp.uint32(sz - 1)), jnp.int32)
            g = plsc.load_gather(buf, [locc], mask=m)
            plsc.store_scatter(g_v, [lanes + base], g, mask=m)

        plsc.parallel_loop(0, _NV, unroll=8)(vbody)

    def accumulate(k):
        def abody(v):
            sl = pl.ds(v * 16, 16)
            g = g_v[sl]
            if k == 0:
                sum_v[sl] = g
                sq_v[sl] = g * g
            else:
                sum_v[sl] = sum_v[sl] + g
                sq_v[sl] = sq_v[sl] + g * g

        plsc.parallel_loop(0, _NV, unroll=8)(abody)

    parks = [None] * _FPT
    cps = [start(0), None]
    for i in range(_NP * _FPT):
        k, p = divmod(i, _NP)
        if i + 1 < _NP * _FPT:
            cps[(i + 1) % 2] = start(i + 1)
        cps[i % 2].wait()
        if p == 0 and k > 0:
            parks[k - 1].wait()
        run_pass(bufs[i % 2], p)
        if p == _NP - 1:
            accumulate(k)
            f = sid * _FPT + k
            parks[k] = pltpu.async_copy(
                g_v, out_hbm.at[f, pl.ds(0, _H)], wsem)

    parks[_FPT - 1].wait()

    f0 = sid * _FPT
    pfs = [pltpu.async_copy(out_hbm.at[f0 + k, pl.ds(0, _H)],
                            buf1.at[pl.ds(k * _H, _H)], psem)
           for k in range(_FPT)]

    pltpu.sync_copy(sum_v, sh_part.at[sid, 0])
    pltpu.sync_copy(sq_v, sh_part.at[sid, 1])
    plsc.subcore_barrier()

    st = sid * _SLICE

    pltpu.sync_copy(sh_part.at[:, 0, pl.ds(st, _SLICE)], part_v)

    def red_sum(v):
        sl = pl.ds(v * 16, 16)
        acc = part_v[0, sl]
        for t in range(1, _NT):
            acc = acc + part_v[t, sl]
        g_v[pl.ds(v * 16, 16)] = acc * inv_d

    plsc.parallel_loop(0, _SLICE // 16, unroll=4)(red_sum)

    pltpu.sync_copy(sh_part.at[:, 1, pl.ds(st, _SLICE)], part_v)

    def red_sq(v):
        sl = pl.ds(v * 16, 16)
        acc = part_v[0, sl]
        for t in range(1, _NT):
            acc = acc + part_v[t, sl]
        mean = g_v[pl.ds(v * 16, 16)]
        var = acc * inv_d - mean * mean + jnp.float32(_EPS)
        g_v[pl.ds(_SLICE + v * 16, 16)] = _rsqrt(var)

    plsc.parallel_loop(0, _SLICE // 16, unroll=4)(red_sq)

    pltpu.sync_copy(g_v.at[pl.ds(0, _SLICE)], sh_stats.at[0, pl.ds(st, _SLICE)])
    pltpu.sync_copy(g_v.at[pl.ds(_SLICE, _SLICE)],
                    sh_stats.at[1, pl.ds(st, _SLICE)])
    plsc.subcore_barrier()

    pltpu.sync_copy(sh_stats.at[0], buf0.at[pl.ds(0, _H)])
    pltpu.sync_copy(sh_stats.at[1], buf0.at[pl.ds(_H, _H)])

    writes = []
    for k in range(_FPT):
        f = sid * _FPT + k
        chunk = (f // 16) * 16
        lane = f - chunk
        gam = _bcast_lane(gam_v[pl.ds(chunk, 16)], lane)
        bet = _bcast_lane(bet_v[pl.ds(chunk, 16)], lane)
        pfs[k].wait()

        def norm(v, k=k, gam=gam, bet=bet):
            sl = pl.ds(k * _H + v * 16, 16)
            mean = buf0[pl.ds(v * 16, 16)]
            rs = buf0[pl.ds(_H + v * 16, 16)]
            buf1[sl] = ((buf1[sl] - mean) * rs) * gam + bet

        plsc.parallel_loop(0, _NV, unroll=8)(norm)
        writes.append(pltpu.async_copy(buf1.at[pl.ds(k * _H, _H)],
                                       out_hbm.at[f, pl.ds(0, _H)], wsem))
    for w in writes:
        w.wait()


def _make_half(bbase):
    mesh = plsc.VectorSubcoreMesh(core_axis_name="c", subcore_axis_name="s",
                                  num_cores=1)
    return pl.kernel(
        functools.partial(_body, bbase),
        mesh=mesh,
        out_type=jax.ShapeDtypeStruct((_D, _H), jnp.float32),
        scratch_types=[
            pltpu.VMEM((_H,), jnp.int32),
            pltpu.VMEM((_BUFW,), jnp.float32),
            pltpu.VMEM((_BUFW,), jnp.float32),
            pltpu.VMEM((_H,), jnp.float32),
            pltpu.VMEM((_H,), jnp.float32),
            pltpu.VMEM((_H,), jnp.float32),
            pltpu.VMEM((_NT, _SLICE), jnp.float32),
            pltpu.VMEM((_D,), jnp.float32),
            pltpu.VMEM((_D,), jnp.float32),
            pltpu.VMEM_SHARED((_NT, 2, _H), jnp.float32),
            pltpu.VMEM_SHARED((2, _H), jnp.float32),
            pltpu.SemaphoreType.DMA,
            pltpu.SemaphoreType.DMA,
            pltpu.SemaphoreType.DMA,
            pltpu.SemaphoreType.DMA,
        ],
        compiler_params=pltpu.CompilerParams(use_tc_tiling_on_sc=False,
                                             needs_layout_passes=False),
    )


def kernel(x, table, gamma, beta):
    xi = x.astype(jnp.int32)
    tt = table.T
    o0 = _make_half(0)(xi, tt, gamma, beta)
    o1 = _make_half(_H)(xi, tt, gamma, beta)
    return jnp.concatenate([o0, o1], axis=1).T


# speedup vs baseline: 1.3754x; 1.3754x over previous
import functools

import jax
import jax.numpy as jnp
from jax import lax
from jax.experimental import pallas as pl
from jax.experimental.pallas import tpu as pltpu
from jax.experimental.pallas import tpu_sc as plsc

_B = 16384
_D = 64
_V = 100000
_EPS = 1e-5
_H = _B // 2
_NT = 16
_FPT = _D // _NT
_NV = _H // 16
_SLICE = _H // _NT
_PASS = ((0, 33336), (33336, 33336), (66672, _V - 66672))
_NP = len(_PASS)
_BUFW = 33336


def _rsqrt(v):
    i = lax.bitcast_convert_type(v, jnp.int32)
    i = jnp.int32(0x5F3759DF) - lax.shift_right_arithmetic(i, 1)
    y = lax.bitcast_convert_type(i, jnp.float32)
    hv = jnp.float32(0.5) * v
    for _ in range(2):
        y = y * (jnp.float32(1.5) - hv * y * y)
    return y


def _bcast_lane(vec, lane):
    idx = jnp.full((16,), lane, dtype=jnp.int32)
    return vec.at[idx].get(mode="promise_in_bounds")


def _body(x_hbm, tab_hbm, gamma_hbm, beta_hbm, out_hbm,
          idx_v, buf0, buf1, g_v, sum_v, sq_v, part_v,
          gam_v, bet_v, sh_part, sh_stats, sem0, sem1, wsem, psem):
    cid = lax.axis_index("c")
    sid = lax.axis_index("s")
    bbase = cid * _H

    pltpu.sync_copy(x_hbm.at[pl.ds(bbase, _H)], idx_v)
    pltpu.sync_copy(gamma_hbm, gam_v)
    pltpu.sync_copy(beta_hbm, bet_v)

    bufs = (buf0, buf1)
    sems = (sem0, sem1)
    lanes = lax.iota(jnp.int32, 16)
    inv_d = jnp.float32(1.0 / _D)

    def start(i):
        k, p = divmod(i, _NP)
        off, sz = _PASS[p]
        f = sid * _FPT + k
        return pltpu.async_copy(tab_hbm.at[f, pl.ds(off, sz)],
                                bufs[i % 2].at[pl.ds(0, sz)], sems[i % 2])

    def run_pass(buf, p):
        off, sz = _PASS[p]
        lo = jnp.int32(off)

        def vbody(v):
            base = v * 16
            iv = idx_v[pl.ds(base, 16)]
            loc = iv - lo
            ul = lax.bitcast_convert_type(loc, jnp.uint32)
            m = ul < jnp.uint32(sz)
            locc = lax.bitcast_convert_type(
                jnp.minimum(ul, jnp.uint32(sz - 1)), jnp.int32)
            g = plsc.load_gather(buf, [locc], mask=m)
            plsc.store_scatter(g_v, [lanes + base], g, mask=m)

        plsc.parallel_loop(0, _NV, unroll=8)(vbody)

    def accumulate(k):
        def abody(v):
            sl = pl.ds(v * 16, 16)
            g = g_v[sl]
            if k == 0:
                sum_v[sl] = g
                sq_v[sl] = g * g
            else:
                sum_v[sl] = sum_v[sl] + g
                sq_v[sl] = sq_v[sl] + g * g

        plsc.parallel_loop(0, _NV, unroll=8)(abody)

    parks = [None] * _FPT
    cps = [start(0), None]
    for i in range(_NP * _FPT):
        k, p = divmod(i, _NP)
        if i + 1 < _NP * _FPT:
            cps[(i + 1) % 2] = start(i + 1)
        cps[i % 2].wait()
        if p == 0 and k > 0:
            parks[k - 1].wait()
        run_pass(bufs[i % 2], p)
        if p == _NP - 1:
            accumulate(k)
            f = sid * _FPT + k
            parks[k] = pltpu.async_copy(
                g_v, out_hbm.at[f, pl.ds(bbase, _H)], wsem)

    parks[_FPT - 1].wait()

    f0 = sid * _FPT
    pfs = [pltpu.async_copy(out_hbm.at[f0 + k, pl.ds(bbase, _H)],
                            buf1.at[pl.ds(k * _H, _H)], psem)
           for k in range(_FPT)]

    pltpu.sync_copy(sum_v, sh_part.at[sid, 0])
    pltpu.sync_copy(sq_v, sh_part.at[sid, 1])
    plsc.subcore_barrier()

    st = sid * _SLICE

    pltpu.sync_copy(sh_part.at[:, 0, pl.ds(st, _SLICE)], part_v)

    def red_sum(v):
        sl = pl.ds(v * 16, 16)
        acc = part_v[0, sl]
        for t in range(1, _NT):
            acc = acc + part_v[t, sl]
        g_v[pl.ds(v * 16, 16)] = acc * inv_d

    plsc.parallel_loop(0, _SLICE // 16, unroll=4)(red_sum)

    pltpu.sync_copy(sh_part.at[:, 1, pl.ds(st, _SLICE)], part_v)

    def red_sq(v):
        sl = pl.ds(v * 16, 16)
        acc = part_v[0, sl]
        for t in range(1, _NT):
            acc = acc + part_v[t, sl]
        mean = g_v[pl.ds(v * 16, 16)]
        var = acc * inv_d - mean * mean + jnp.float32(_EPS)
        g_v[pl.ds(_SLICE + v * 16, 16)] = _rsqrt(var)

    plsc.parallel_loop(0, _SLICE // 16, unroll=4)(red_sq)

    pltpu.sync_copy(g_v.at[pl.ds(0, _SLICE)], sh_stats.at[0, pl.ds(st, _SLICE)])
    pltpu.sync_copy(g_v.at[pl.ds(_SLICE, _SLICE)],
                    sh_stats.at[1, pl.ds(st, _SLICE)])
    plsc.subcore_barrier()

    pltpu.sync_copy(sh_stats.at[0], buf0.at[pl.ds(0, _H)])
    pltpu.sync_copy(sh_stats.at[1], buf0.at[pl.ds(_H, _H)])

    writes = []
    for k in range(_FPT):
        f = sid * _FPT + k
        chunk = (f // 16) * 16
        lane = f - chunk
        gam = _bcast_lane(gam_v[pl.ds(chunk, 16)], lane)
        bet = _bcast_lane(bet_v[pl.ds(chunk, 16)], lane)
        pfs[k].wait()

        def norm(v, k=k, gam=gam, bet=bet):
            sl = pl.ds(k * _H + v * 16, 16)
            mean = buf0[pl.ds(v * 16, 16)]
            rs = buf0[pl.ds(_H + v * 16, 16)]
            buf1[sl] = ((buf1[sl] - mean) * rs) * gam + bet

        plsc.parallel_loop(0, _NV, unroll=8)(norm)
        writes.append(pltpu.async_copy(buf1.at[pl.ds(k * _H, _H)],
                                       out_hbm.at[f, pl.ds(bbase, _H)], wsem))
    for w in writes:
        w.wait()


def kernel(x, table, gamma, beta):
    mesh = plsc.VectorSubcoreMesh(core_axis_name="c", subcore_axis_name="s")
    f = pl.kernel(
        _body,
        mesh=mesh,
        out_type=jax.ShapeDtypeStruct((_D, _B), jnp.float32),
        scratch_types=[
            pltpu.VMEM((_H,), jnp.int32),
            pltpu.VMEM((_BUFW,), jnp.float32),
            pltpu.VMEM((_BUFW,), jnp.float32),
            pltpu.VMEM((_H,), jnp.float32),
            pltpu.VMEM((_H,), jnp.float32),
            pltpu.VMEM((_H,), jnp.float32),
            pltpu.VMEM((_NT, _SLICE), jnp.float32),
            pltpu.VMEM((_D,), jnp.float32),
            pltpu.VMEM((_D,), jnp.float32),
            pltpu.VMEM_SHARED((_NT, 2, _H), jnp.float32),
            pltpu.VMEM_SHARED((2, _H), jnp.float32),
            pltpu.SemaphoreType.DMA,
            pltpu.SemaphoreType.DMA,
            pltpu.SemaphoreType.DMA,
            pltpu.SemaphoreType.DMA,
        ],
        compiler_params=pltpu.CompilerParams(use_tc_tiling_on_sc=False,
                                             needs_layout_passes=False),
    )
    out_t = f(x.astype(jnp.int32), table.T, gamma, beta)
    return out_t.T
